# speedup vs baseline: 2.4463x; 2.4463x over previous
import functools

import jax
import jax.numpy as jnp
from jax.experimental import pallas as pl

_TOPK = 2
_N_EXPERTS = 64
_D_MODEL = 768
_BLOCK_TOKENS = 2048


def _router_kernel(x_ref, w_ref, wout_ref, iout_ref):
    x = x_ref[...]
    w = w_ref[...]
    scores = jax.lax.dot_general(
        x, w, (((1,), (1,)), ((), ())),
        preferred_element_type=jnp.float32)

    m = jnp.max(scores, axis=-1, keepdims=True)
    e = jnp.exp(scores - m)
    denom = jnp.sum(e, axis=-1, keepdims=True)

    idx = jax.lax.broadcasted_iota(jnp.int32, scores.shape, 1)
    big = jnp.int32(_N_EXPERTS)
    i1 = jnp.min(jnp.where(scores == m, idx, big), axis=-1, keepdims=True)
    masked = jnp.where(idx == i1, -jnp.inf, scores)
    v2 = jnp.max(masked, axis=-1, keepdims=True)
    i2 = jnp.min(jnp.where(masked == v2, idx, big), axis=-1, keepdims=True)

    w1 = 1.0 / denom
    w2 = jnp.exp(v2 - m) / denom
    wout_ref[...] = jnp.concatenate([w1, w2], axis=-1)
    iout_ref[...] = jnp.concatenate([i1, i2], axis=-1)


@jax.jit
def kernel(x, W):
    n_tokens, d_model = x.shape
    n_experts = W.shape[0]
    b = _BLOCK_TOKENS
    grid = (n_tokens // b,)
    weights, indices = pl.pallas_call(
        _router_kernel,
        grid=grid,
        in_specs=[
            pl.BlockSpec((b, d_model), lambda i: (i, 0)),
            pl.BlockSpec((n_experts, d_model), lambda i: (0, 0)),
        ],
        out_specs=[
            pl.BlockSpec((b, _TOPK), lambda i: (i, 0)),
            pl.BlockSpec((b, _TOPK), lambda i: (i, 0)),
        ],
        out_shape=[
            jax.ShapeDtypeStruct((n_tokens, _TOPK), jnp.float32),
            jax.ShapeDtypeStruct((n_tokens, _TOPK), jnp.int32),
        ],
    )(x, W)
    return (weights, indices)
